# Initial kernel scaffold; baseline (speedup 1.0000x reference)
import jax
import jax.numpy as jnp
from jax.experimental import pallas as pl


def kernel(z, W_q, b_q, codebook, W_pq, b_pq):
    raise NotImplementedError("write your pallas kernel here")


# speedup vs baseline: 1.1699x; 1.1699x over previous
import jax
import jax.numpy as jnp
from jax.experimental import pallas as pl
from jax.experimental.pallas import tpu as pltpu
from jax.experimental.pallas import tpu_sc as plsc

_DIM = 768
_CDIM = 256
_K = 8192
_TOK_TILE = 256
_GATHER_WINDOW = 128


def _argmin_kernel(z_ref, wq_ref, bq_ref, cb_ref, idx_ref, dsum_ref):
    zp = jax.lax.dot_general(
        z_ref[...], wq_ref[...], (((1,), (1,)), ((), ()))) + bq_ref[...]
    zp2 = jnp.sum(zp ** 2, axis=1, keepdims=True)
    cb = cb_ref[...]
    cb2 = jnp.sum(cb ** 2, axis=1)[None, :]
    mm = jax.lax.dot_general(zp, cb, (((1,), (1,)), ((), ())))
    d = (zp2 + cb2) - 2.0 * mm
    dmin = jnp.min(d, axis=1, keepdims=True)
    iota = jax.lax.broadcasted_iota(jnp.int32, d.shape, 1)
    idx = jnp.min(jnp.where(d == dmin, iota, _K), axis=1)
    idx_ref[0, 0, :] = idx.astype(jnp.int32)

    @pl.when(pl.program_id(0) == 0)
    def _():
        dsum_ref[...] = jnp.zeros((1, 1), jnp.float32)

    dsum_ref[...] += jnp.sum(dmin).reshape(1, 1)


def _proj_kernel(zq_ref, wpq_ref, bpq_ref, out_ref):
    out_ref[...] = jax.lax.dot_general(
        zq_ref[...], wpq_ref[...], (((1,), (1,)), ((), ()))) + bpq_ref[...]


def _sc_gather(table, idx_row):
    n_rows = idx_row.shape[1]
    width = table.shape[1]
    mesh = plsc.VectorSubcoreMesh(core_axis_name="core",
                                  subcore_axis_name="subcore")

    @pl.kernel(out_type=jax.ShapeDtypeStruct((n_rows, width), table.dtype),
               mesh=mesh)
    def kern(cb_hbm, i_hbm, o_hbm):
        def body(i_vmem, o_vmem):
            pltpu.sync_copy(cb_hbm.at[i_vmem.at[0]], o_vmem)

        pltpu.emit_pipeline(
            body,
            grid=(n_rows // _GATHER_WINDOW,),
            in_specs=[pl.BlockSpec((1, _GATHER_WINDOW),
                                   index_map=lambda i: (0, i))],
            out_specs=[pl.BlockSpec((_GATHER_WINDOW, width),
                                    index_map=lambda i: (i, 0))],
            core_axis_name=("core", "subcore"),
            dimension_semantics=(pltpu.PARALLEL,),
        )(i_hbm, o_hbm)

    return kern(table, idx_row)


def kernel(z, W_q, b_q, codebook, W_pq, b_pq):
    B, HW, dim = z.shape
    n = B * HW
    zf = z.reshape(n, dim)
    n_tiles = n // _TOK_TILE

    idx3, dsum = pl.pallas_call(
        _argmin_kernel,
        grid=(n_tiles,),
        in_specs=[
            pl.BlockSpec((_TOK_TILE, _DIM), lambda i: (i, 0)),
            pl.BlockSpec((_CDIM, _DIM), lambda i: (0, 0)),
            pl.BlockSpec((1, _CDIM), lambda i: (0, 0)),
            pl.BlockSpec((_K, _CDIM), lambda i: (0, 0)),
        ],
        out_specs=[
            pl.BlockSpec((1, 1, _TOK_TILE), lambda i: (i, 0, 0)),
            pl.BlockSpec((1, 1), lambda i: (0, 0)),
        ],
        out_shape=[
            jax.ShapeDtypeStruct((n_tiles, 1, _TOK_TILE), jnp.int32),
            jax.ShapeDtypeStruct((1, 1), jnp.float32),
        ],
    )(zf, W_q, b_q.reshape(1, _CDIM), codebook)

    idx_flat = idx3.reshape(n)
    idx2 = (2 * idx_flat[:, None] + jnp.arange(2, dtype=jnp.int32))
    zq = _sc_gather(codebook.reshape(2 * _K, 128),
                    idx2.reshape(1, 2 * n)).reshape(n, _CDIM)

    out = pl.pallas_call(
        _proj_kernel,
        grid=(B,),
        in_specs=[
            pl.BlockSpec((HW, _CDIM), lambda i: (i, 0)),
            pl.BlockSpec((_DIM, _CDIM), lambda i: (0, 0)),
            pl.BlockSpec((1, _DIM), lambda i: (0, 0)),
        ],
        out_specs=pl.BlockSpec((HW, _DIM), lambda i: (i, 0)),
        out_shape=jax.ShapeDtypeStruct((n, _DIM), jnp.float32),
    )(zq, W_pq, b_pq.reshape(1, _DIM))

    loss = dsum[0, 0] * (1.25 / (n * _CDIM))
    return out.reshape(B, HW, dim), idx_flat.reshape(B, HW), loss


# speedup vs baseline: 1.4424x; 1.2329x over previous
import jax
import jax.numpy as jnp
from jax.experimental import pallas as pl
from jax.experimental.pallas import tpu as pltpu
from jax.experimental.pallas import tpu_sc as plsc

_DIM = 768
_CDIM = 256
_K = 8192
_TOK_TILE = 256
_GATHER_WINDOW = 128


def _argmin_kernel(z_ref, wq_hbm, bq_ref, cb_hbm, idx_ref, dsum_ref,
                   cb_ref, wq_ref, cb2_ref, sem):
    @pl.when(pl.program_id(0) == 0)
    def _():
        cp1 = pltpu.make_async_copy(cb_hbm, cb_ref, sem)
        cp1.start()
        cp2 = pltpu.make_async_copy(wq_hbm, wq_ref, sem)
        cp2.start()
        cp1.wait()
        cp2.wait()
        cb2_ref[...] = jax.lax.dot_general(
            jnp.ones((1, _CDIM), jnp.float32), cb_ref[...] ** 2,
            (((1,), (1,)), ((), ())))
        dsum_ref[...] = jnp.zeros((1, 1), jnp.float32)

    zp = jax.lax.dot_general(
        z_ref[...], wq_ref[...], (((1,), (1,)), ((), ()))) + bq_ref[...]
    zp2 = jnp.sum(zp ** 2, axis=1, keepdims=True)
    mm2n = jax.lax.dot_general(
        -2.0 * zp, cb_ref[...], (((1,), (1,)), ((), ())))
    d = (zp2 + cb2_ref[...]) + mm2n
    dmin = jnp.min(d, axis=1, keepdims=True)
    iota = jax.lax.broadcasted_iota(jnp.int32, d.shape, 1)
    idx = jnp.min(jnp.where(d == dmin, iota, _K), axis=1)
    idx_ref[0, 0, :] = idx.astype(jnp.int32)

    dsum_ref[...] += jnp.sum(dmin).reshape(1, 1)


def _proj_kernel(zq_ref, wpq_ref, bpq_ref, out_ref):
    out_ref[...] = jax.lax.dot_general(
        zq_ref[...], wpq_ref[...], (((1,), (1,)), ((), ()))) + bpq_ref[...]


def _sc_gather(table, idx_row):
    n_rows = idx_row.shape[1]
    width = table.shape[1]
    mesh = plsc.VectorSubcoreMesh(core_axis_name="core",
                                  subcore_axis_name="subcore")

    @pl.kernel(out_type=jax.ShapeDtypeStruct((n_rows, width), table.dtype),
               mesh=mesh)
    def kern(cb_hbm, i_hbm, o_hbm):
        def body(i_vmem, o_vmem):
            pltpu.sync_copy(cb_hbm.at[i_vmem.at[0]], o_vmem)

        pltpu.emit_pipeline(
            body,
            grid=(n_rows // _GATHER_WINDOW,),
            in_specs=[pl.BlockSpec((1, _GATHER_WINDOW),
                                   index_map=lambda i: (0, i))],
            out_specs=[pl.BlockSpec((_GATHER_WINDOW, width),
                                    index_map=lambda i: (i, 0))],
            core_axis_name=("core", "subcore"),
            dimension_semantics=(pltpu.PARALLEL,),
        )(i_hbm, o_hbm)

    return kern(table, idx_row)


def kernel(z, W_q, b_q, codebook, W_pq, b_pq):
    B, HW, dim = z.shape
    n = B * HW
    zf = z.reshape(n, dim)
    n_tiles = n // _TOK_TILE

    idx3, dsum = pl.pallas_call(
        _argmin_kernel,
        grid=(n_tiles,),
        in_specs=[
            pl.BlockSpec((_TOK_TILE, _DIM), lambda i: (i, 0)),
            pl.BlockSpec(memory_space=pl.ANY),
            pl.BlockSpec((1, _CDIM), lambda i: (0, 0)),
            pl.BlockSpec(memory_space=pl.ANY),
        ],
        out_specs=[
            pl.BlockSpec((1, 1, _TOK_TILE), lambda i: (i, 0, 0)),
            pl.BlockSpec((1, 1), lambda i: (0, 0)),
        ],
        out_shape=[
            jax.ShapeDtypeStruct((n_tiles, 1, _TOK_TILE), jnp.int32),
            jax.ShapeDtypeStruct((1, 1), jnp.float32),
        ],
        scratch_shapes=[
            pltpu.VMEM((_K, _CDIM), jnp.float32),
            pltpu.VMEM((_CDIM, _DIM), jnp.float32),
            pltpu.VMEM((1, _K), jnp.float32),
            pltpu.SemaphoreType.DMA,
        ],
    )(zf, W_q, b_q.reshape(1, _CDIM), codebook)

    idx_flat = idx3.reshape(n)
    zq = _sc_gather(codebook, idx3.reshape(1, n))

    out = pl.pallas_call(
        _proj_kernel,
        grid=(B,),
        in_specs=[
            pl.BlockSpec((HW, _CDIM), lambda i: (i, 0)),
            pl.BlockSpec((_DIM, _CDIM), lambda i: (0, 0)),
            pl.BlockSpec((1, _DIM), lambda i: (0, 0)),
        ],
        out_specs=pl.BlockSpec((HW, _DIM), lambda i: (i, 0)),
        out_shape=jax.ShapeDtypeStruct((n, _DIM), jnp.float32),
    )(zq, W_pq, b_pq.reshape(1, _DIM))

    loss = dsum[0, 0] * (1.25 / (n * _CDIM))
    return out.reshape(B, HW, dim), idx_flat.reshape(B, HW), loss


# speedup vs baseline: 1.4548x; 1.0086x over previous
import jax
import jax.numpy as jnp
from jax.experimental import pallas as pl
from jax.experimental.pallas import tpu as pltpu
from jax.experimental.pallas import tpu_sc as plsc

_DIM = 768
_CDIM = 256
_K = 8192
_TOK_TILE = 512
_GATHER_WINDOW = 128


def _argmin_kernel(z_ref, wq_hbm, bq_ref, cb_hbm, idx_ref, dsum_ref,
                   cb_ref, wq_ref, cb2_ref, sem):
    @pl.when(pl.program_id(0) == 0)
    def _():
        cp1 = pltpu.make_async_copy(cb_hbm, cb_ref, sem)
        cp1.start()
        cp2 = pltpu.make_async_copy(wq_hbm, wq_ref, sem)
        cp2.start()
        cp1.wait()
        cp2.wait()
        cb2_ref[...] = jax.lax.dot_general(
            jnp.ones((1, _CDIM), jnp.float32), cb_ref[...] ** 2,
            (((1,), (1,)), ((), ())))
        dsum_ref[...] = jnp.zeros((1, 1), jnp.float32)

    zp = jax.lax.dot_general(
        z_ref[...], wq_ref[...], (((1,), (1,)), ((), ()))) + bq_ref[...]
    zp2 = jnp.sum(zp ** 2, axis=1, keepdims=True)
    mm2n = jax.lax.dot_general(
        -2.0 * zp, cb_ref[...], (((1,), (1,)), ((), ())))
    d = (zp2 + cb2_ref[...]) + mm2n
    dmin = jnp.min(d, axis=1, keepdims=True)
    iota = jax.lax.broadcasted_iota(jnp.int32, d.shape, 1)
    idx = jnp.min(jnp.where(d == dmin, iota, _K), axis=1)
    idx_ref[0, 0, :] = idx.astype(jnp.int32)

    dsum_ref[...] += jnp.sum(dmin).reshape(1, 1)


def _proj_kernel(zq_ref, wpq_ref, bpq_ref, out_ref):
    out_ref[...] = jax.lax.dot_general(
        zq_ref[...], wpq_ref[...], (((1,), (1,)), ((), ()))) + bpq_ref[...]


def _sc_gather(table, idx_row):
    n_rows = idx_row.shape[1]
    width = table.shape[1]
    mesh = plsc.VectorSubcoreMesh(core_axis_name="core",
                                  subcore_axis_name="subcore")

    @pl.kernel(out_type=jax.ShapeDtypeStruct((n_rows, width), table.dtype),
               mesh=mesh)
    def kern(cb_hbm, i_hbm, o_hbm):
        def body(i_vmem, o_vmem):
            pltpu.sync_copy(cb_hbm.at[i_vmem.at[0]], o_vmem)

        pltpu.emit_pipeline(
            body,
            grid=(n_rows // _GATHER_WINDOW,),
            in_specs=[pl.BlockSpec((1, _GATHER_WINDOW),
                                   index_map=lambda i: (0, i))],
            out_specs=[pl.BlockSpec((_GATHER_WINDOW, width),
                                    index_map=lambda i: (i, 0))],
            core_axis_name=("core", "subcore"),
            dimension_semantics=(pltpu.PARALLEL,),
        )(i_hbm, o_hbm)

    return kern(table, idx_row)


def kernel(z, W_q, b_q, codebook, W_pq, b_pq):
    B, HW, dim = z.shape
    n = B * HW
    zf = z.reshape(n, dim)
    n_tiles = n // _TOK_TILE

    idx3, dsum = pl.pallas_call(
        _argmin_kernel,
        grid=(n_tiles,),
        in_specs=[
            pl.BlockSpec((_TOK_TILE, _DIM), lambda i: (i, 0)),
            pl.BlockSpec(memory_space=pl.ANY),
            pl.BlockSpec((1, _CDIM), lambda i: (0, 0)),
            pl.BlockSpec(memory_space=pl.ANY),
        ],
        out_specs=[
            pl.BlockSpec((1, 1, _TOK_TILE), lambda i: (i, 0, 0)),
            pl.BlockSpec((1, 1), lambda i: (0, 0)),
        ],
        out_shape=[
            jax.ShapeDtypeStruct((n_tiles, 1, _TOK_TILE), jnp.int32),
            jax.ShapeDtypeStruct((1, 1), jnp.float32),
        ],
        scratch_shapes=[
            pltpu.VMEM((_K, _CDIM), jnp.float32),
            pltpu.VMEM((_CDIM, _DIM), jnp.float32),
            pltpu.VMEM((1, _K), jnp.float32),
            pltpu.SemaphoreType.DMA,
        ],
    )(zf, W_q, b_q.reshape(1, _CDIM), codebook)

    idx_flat = idx3.reshape(n)
    zq = _sc_gather(codebook, idx3.reshape(1, n))

    out = pl.pallas_call(
        _proj_kernel,
        grid=(B,),
        in_specs=[
            pl.BlockSpec((HW, _CDIM), lambda i: (i, 0)),
            pl.BlockSpec((_DIM, _CDIM), lambda i: (0, 0)),
            pl.BlockSpec((1, _DIM), lambda i: (0, 0)),
        ],
        out_specs=pl.BlockSpec((HW, _DIM), lambda i: (i, 0)),
        out_shape=jax.ShapeDtypeStruct((n, _DIM), jnp.float32),
    )(zq, W_pq, b_pq.reshape(1, _DIM))

    loss = dsum[0, 0] * (1.25 / (n * _CDIM))
    return out.reshape(B, HW, dim), idx_flat.reshape(B, HW), loss
